# Initial kernel scaffold; baseline (speedup 1.0000x reference)
import jax
import jax.numpy as jnp
from jax.experimental import pallas as pl


def kernel(node_features, distance, params, edge_index):
    raise NotImplementedError("write your pallas kernel here")


# speedup vs baseline: 1.7252x; 1.7252x over previous
import functools

import jax
import jax.numpy as jnp
from jax import lax
from jax.experimental import pallas as pl
from jax.experimental.pallas import tpu as pltpu
from jax.experimental.pallas import tpu_sc as plsc

N = 10000
E = 160000
D = 128
N_ITERS = 10

NC = 2
NS = 16
NW = NC * NS
E_PAD = 163840
BPW = E_PAD // NW
KJ = BPW // 128
N_PAD = 10240
ROWS_PT = N_PAD // NS

BE = 4096
BN = 2000

@functools.lru_cache(maxsize=None)
def _mesh():
    return plsc.VectorSubcoreMesh(core_axis_name="c", subcore_axis_name="s",
                                  num_cores=NC, num_subcores=NS)


def _gather_body(table, idx, out, idx_v, rows_v, sem):
    c = lax.axis_index("c")
    s = lax.axis_index("s")
    wid = s * NC + c
    pltpu.sync_copy(idx.at[wid], idx_v)

    def chunk(j, carry):
        pltpu.async_copy(table.at[idx_v.at[j]], rows_v, sem).wait()
        pltpu.sync_copy(rows_v, out.at[pl.ds(wid * BPW + j * 128, 128)])
        return carry

    lax.fori_loop(0, KJ, chunk, 0)


@functools.lru_cache(maxsize=None)
def _gather_kernel(dt):
    return pl.kernel(
        _gather_body,
        out_type=jax.ShapeDtypeStruct((E_PAD, dt), jnp.float32),
        mesh=_mesh(),
        scratch_types=[
            pltpu.VMEM((KJ, 128), jnp.int32),
            pltpu.VMEM((128, dt), jnp.float32),
            pltpu.SemaphoreType.DMA,
        ],
    )


def _sc_gather(table, idx3):
    return _gather_kernel(table.shape[1])(table, idx3)


def _scatter_body(vals, idx, zeros, out, idx_v, rows_v, acc):
    c = lax.axis_index("c")
    s = lax.axis_index("s")
    wid = s * NC + c
    pltpu.sync_copy(idx.at[wid], idx_v)
    pltpu.sync_copy(zeros.at[pl.ds(s * ROWS_PT, ROWS_PT)],
                    acc.at[pl.ds(s * ROWS_PT, ROWS_PT)])
    plsc.subcore_barrier()

    def chunk(j, carry):
        pltpu.sync_copy(vals.at[pl.ds(wid * BPW + j * 128, 128)], rows_v)
        pltpu.sync_copy(rows_v, acc.at[idx_v.at[j]], add=True)
        return carry

    lax.fori_loop(0, KJ, chunk, 0)
    plsc.subcore_barrier()
    pltpu.sync_copy(acc.at[pl.ds(s * ROWS_PT, ROWS_PT)],
                    out.at[c, pl.ds(s * ROWS_PT, ROWS_PT)])


@functools.lru_cache(maxsize=None)
def _scatter_kernel():
    return pl.kernel(
        _scatter_body,
        out_type=jax.ShapeDtypeStruct((NC, N_PAD, D), jnp.float32),
        mesh=_mesh(),
        scratch_types=[
            pltpu.VMEM((KJ, 128), jnp.int32),
            pltpu.VMEM((128, D), jnp.float32),
            pltpu.VMEM_SHARED((N_PAD, D), jnp.float32),
        ],
    )


def _sc_scatter(vals, idx3, zeros):
    return _scatter_kernel()(vals, idx3, zeros)


def _dot(a, b):
    return jnp.dot(a, b, preferred_element_type=jnp.float32)


def _b16(a):
    return a.astype(jnp.bfloat16).astype(jnp.float32)


_W = pl.BlockSpec((D, D), lambda i: (0, 0))
_B = pl.BlockSpec((1, D), lambda i: (0, 0))
_S = pl.BlockSpec((1, 1), lambda i: (0, 0))


def _edge_mlp_body(g_ref, s_ref, w2_ref, b2_ref, w3_ref, b3_ref, o_ref):
    i = pl.program_id(0)
    g = jnp.maximum(g_ref[...] + s_ref[...], 0.0)
    t = jnp.maximum(_dot(g, w2_ref[...]) + b2_ref[...], 0.0)
    o = _dot(t, w3_ref[...]) + b3_ref[...]
    rows = i * BE + lax.broadcasted_iota(jnp.int32, (BE, 1), 0)
    o_ref[...] = jnp.where(rows < E, o, 0.0)


def _edge_mlp(g, ss, w2, b2, w3, b3):
    eb = pl.BlockSpec((BE, D), lambda i: (i, 0))
    return pl.pallas_call(
        _edge_mlp_body,
        grid=(E_PAD // BE,),
        in_specs=[eb, eb, _W, _B, _W, _B],
        out_specs=eb,
        out_shape=jax.ShapeDtypeStruct((E_PAD, D), jnp.float32),
    )(g, ss, w2, b2, w3, b3)


def _edge_clf_body(g_ref, s_ref, v1_ref, c1_ref, v2t_ref, c2_ref, ep_ref,
                   o_ref):
    g = jnp.maximum(g_ref[...] + s_ref[...], 0.0)
    t = jnp.maximum(_dot(g, v1_ref[...]) + c1_ref[...], 0.0)
    val = (jnp.sum(_b16(t) * _b16(v2t_ref[...]), axis=1, keepdims=True)
           + c2_ref[...])
    o_ref[...] = ep_ref[...] + val


def _edge_clf(g, ss, v1, c1, v2t, c2, ep):
    eb1 = pl.BlockSpec((BE, D), lambda i: (i, 1))
    ev = pl.BlockSpec((BE, 1), lambda i: (i, 0))
    return pl.pallas_call(
        _edge_clf_body,
        grid=(E_PAD // BE,),
        in_specs=[eb1, eb1, _W, _B, _B, _S, ev],
        out_specs=ev,
        out_shape=jax.ShapeDtypeStruct((E_PAD, 1), jnp.float32),
    )(g, ss, v1, c1, v2t, c2, ep)


def _node_body(m_ref, h_ref, xn_ref, a1_ref, a2_ref, np_ref,
               wm_ref, wh_ref, w1_ref, b1_ref, w2_ref, b2_ref,
               u0_ref, d0_ref, u1t_ref, d1_ref,
               wb1_ref, wd1_ref, vb1_ref, vd1_ref,
               h_o, cc_o, np_o):
    msg = m_ref[0] + m_ref[1]
    t = jnp.maximum(_dot(msg, wm_ref[...]) + _dot(h_ref[...], wh_ref[...])
                    + xn_ref[...], 0.0)
    t = jnp.maximum(_dot(t, w1_ref[...]) + b1_ref[...], 0.0)
    hn = _dot(t, w2_ref[...]) + b2_ref[...]
    u = jnp.maximum(_dot(hn, u0_ref[...]) + d0_ref[...], 0.0)
    npv = (jnp.sum(_b16(u) * _b16(u1t_ref[...]), axis=1, keepdims=True)
           + d1_ref[...])
    h_o[...] = hn
    cc_o[...] = jnp.concatenate(
        [a1_ref[...] + _dot(hn, wb1_ref[...]) + _dot(hn, wd1_ref[...]),
         a2_ref[...] + _dot(hn, vb1_ref[...]) + _dot(hn, vd1_ref[...])],
        axis=1)
    np_o[...] = np_ref[...] + npv


def _node_update(mp, h, xn, a1, a2, npd, wm, wh, w1, b1, w2, b2,
                 u0, d0, u1t, dd1, wb1, wd1, vb1, vd1):
    nb = pl.BlockSpec((BN, D), lambda i: (i, 0))
    nv = pl.BlockSpec((BN, 1), lambda i: (i, 0))
    return pl.pallas_call(
        _node_body,
        grid=(N // BN,),
        in_specs=[pl.BlockSpec((NC, BN, D), lambda i: (0, i, 0)),
                  nb, nb, nb, nb, nv,
                  _W, _W, _W, _B, _W, _B, _W, _B, _B, _S, _W, _W, _W, _W],
        out_specs=[nb, pl.BlockSpec((BN, 2 * D), lambda i: (i, 0)), nv],
        out_shape=[jax.ShapeDtypeStruct((N, D), jnp.float32),
                   jax.ShapeDtypeStruct((N, 2 * D), jnp.float32),
                   jax.ShapeDtypeStruct((N, 1), jnp.float32)],
    )(mp, h, xn, a1, a2, npd, wm, wh, w1, b1, w2, b2,
      u0, d0, u1t, dd1, wb1, wd1, vb1, vd1)


def _pre_body(x_ref, p0, pb0, p1, pb1, p2, pb2,
              w0a, w0c, v0a, v0c, wnx, bn0, wb1, wd1,
              h_o, c1_o, a1_o, a2_o, bxs_o, xn_o):
    x = x_ref[...]
    t = jnp.maximum(_dot(x, p0[...]) + pb0[...], 0.0)
    t = jnp.maximum(_dot(t, p1[...]) + pb1[...], 0.0)
    h0 = _dot(t, p2[...]) + pb2[...]
    a1 = _dot(x, w0a[...])
    a2 = _dot(x, v0a[...])
    h_o[...] = h0
    c1_o[...] = a1 + _dot(h0, wb1[...]) + _dot(h0, wd1[...])
    a1_o[...] = a1
    a2_o[...] = a2
    bxs_o[...] = jnp.concatenate([_dot(x, w0c[...]), _dot(x, v0c[...])],
                                 axis=1)
    xn_o[...] = _dot(x, wnx[...]) + bn0[...]


def _precompute(x, p0, pb0, p1, pb1, p2, pb2, w0a, w0c, v0a, v0c, wnx, bn0,
                wb1, wd1):
    nb = pl.BlockSpec((BN, D), lambda i: (i, 0))
    return pl.pallas_call(
        _pre_body,
        grid=(N // BN,),
        in_specs=[nb, _W, _B, _W, _B, _W, _B, _W, _W, _W, _W, _W, _B, _W,
                  _W],
        out_specs=[nb, nb, nb, nb, pl.BlockSpec((BN, 2 * D), lambda i: (i, 0)),
                   nb],
        out_shape=[jax.ShapeDtypeStruct((N, D), jnp.float32)] * 4
        + [jax.ShapeDtypeStruct((N, 2 * D), jnp.float32),
           jax.ShapeDtypeStruct((N, D), jnp.float32)],
    )(x, p0, pb0, p1, pb1, p2, pb2, w0a, w0c, v0a, v0c, wnx, bn0, wb1, wd1)


def _sassy_body(gs_ref, d_ref, wd1, b0, wd2, c0, ss_o):
    d = _b16(d_ref[...])
    s1 = gs_ref[:, :D] + d * _b16(wd1[...]) + b0[...]
    s2 = gs_ref[:, D:] + d * _b16(wd2[...]) + c0[...]
    ss_o[...] = jnp.concatenate([s1, s2], axis=1)


def _s_assembly(gsrc, d1, wd1, b0, wd2, c0):
    eb2 = pl.BlockSpec((BE, 2 * D), lambda i: (i, 0))
    ev = pl.BlockSpec((BE, 1), lambda i: (i, 0))
    return pl.pallas_call(
        _sassy_body,
        grid=(E_PAD // BE,),
        in_specs=[eb2, ev, _B, _B, _B, _B],
        out_specs=eb2,
        out_shape=jax.ShapeDtypeStruct((E_PAD, 2 * D), jnp.float32),
    )(gsrc, d1, wd1, b0, wd2, c0)


def kernel(node_features, distance, params, edge_index):
    x = node_features.astype(jnp.float32)
    src = edge_index[0].astype(jnp.int32)
    dst = edge_index[1].astype(jnp.int32)
    pad = E_PAD - E
    srcp = jnp.concatenate([src, jnp.zeros((pad,), jnp.int32)]) \
        .reshape(NW, KJ, 128)
    dstp = jnp.concatenate([dst, jnp.zeros((pad,), jnp.int32)]) \
        .reshape(NW, KJ, 128)
    d1 = jnp.concatenate([distance.astype(jnp.float32),
                          jnp.zeros((pad,), jnp.float32)]).reshape(E_PAD, 1)

    en = params['edge_net']
    nn = params['node_net']
    ec = params['edge_clf']
    ni = params['node_init']
    ncf = params['node_clf']

    w0 = en[0]['W']
    w0a, w0c = w0[0:D], w0[2 * D:3 * D]
    wb1, wdd1 = w0[D:2 * D], w0[3 * D:4 * D]
    wd1 = w0[4 * D:4 * D + 1]
    b0 = en[0]['b'].reshape(1, D)
    w2, b2 = en[1]['W'], en[1]['b'].reshape(1, D)
    w3, b3 = en[2]['W'], en[2]['b'].reshape(1, D)

    v0 = ec[0]['W']
    v0a, v0c = v0[0:D], v0[2 * D:3 * D]
    vb1, vdd1 = v0[D:2 * D], v0[3 * D:4 * D]
    wd2 = v0[4 * D:4 * D + 1]
    c0 = ec[0]['b'].reshape(1, D)
    v1, c1 = ec[1]['W'], ec[1]['b'].reshape(1, D)
    v2t = ec[2]['W'].reshape(1, D)
    c2 = ec[2]['b'].reshape(1, 1)

    wn0 = nn[0]['W']
    wm, wnx, wh = wn0[0:D], wn0[D:2 * D], wn0[2 * D:3 * D]
    bn0 = nn[0]['b'].reshape(1, D)
    wn1, bn1 = nn[1]['W'], nn[1]['b'].reshape(1, D)
    wn2, bn2 = nn[2]['W'], nn[2]['b'].reshape(1, D)

    u0, d0 = ncf[0]['W'], ncf[0]['b'].reshape(1, D)
    u1t = ncf[1]['W'].reshape(1, D)
    dd1 = ncf[1]['b'].reshape(1, 1)

    zeros_n = jnp.zeros((N_PAD, D), jnp.float32)
    ep = jnp.zeros((E_PAD, 1), jnp.float32)
    npd = jnp.zeros((N, 1), jnp.float32)

    h, c10, a1, a2, bxs, xn = _precompute(
        x, ni[0]['W'], ni[0]['b'].reshape(1, D), ni[1]['W'],
        ni[1]['b'].reshape(1, D), ni[2]['W'], ni[2]['b'].reshape(1, D),
        w0a, w0c, v0a, v0c, wnx, bn0, wb1, wdd1)
    gsrc = _sc_gather(bxs, srcp)
    ss = _s_assembly(gsrc, d1, wd1, b0, wd2, c0)
    gcur = _sc_gather(c10, dstp)

    for _ in range(N_ITERS):
        e3 = _edge_mlp(gcur, ss, w2, b2, w3, b3)
        mp = _sc_scatter(e3, dstp, zeros_n)
        h, cc, npd = _node_update(mp, h, xn, a1, a2, npd, wm, wh, wn1, bn1,
                                  wn2, bn2, u0, d0, u1t, dd1,
                                  wb1, wdd1, vb1, vdd1)
        gcur = _sc_gather(cc, dstp)
        ep = _edge_clf(gcur, ss, v1, c1, v2t, c2, ep)

    return h, ep[:E, 0], npd[:, 0]


# speedup vs baseline: 3.1016x; 1.7979x over previous
import functools

import jax
import jax.numpy as jnp
from jax import lax
from jax.experimental import pallas as pl
from jax.experimental.pallas import tpu as pltpu
from jax.experimental.pallas import tpu_sc as plsc

N = 10000
E = 160000
D = 128
N_ITERS = 10

NC = 2
NS = 16
NW = NC * NS
E_PAD = 163840
BPW = E_PAD // NW
KJ = BPW // 128
KJG = E_PAD // NS // 128
NBUF = 2
N_PAD = 10240
ROWS_PT = N_PAD // NS

BE = 4096
BN = 2000

@functools.lru_cache(maxsize=None)
def _mesh():
    return plsc.VectorSubcoreMesh(core_axis_name="c", subcore_axis_name="s",
                                  num_cores=NC, num_subcores=NS)


def _gather_body(table, idx, out, idx_v, b0, b1, stg, g0, g1, o0, o1):
    c = lax.axis_index("c")
    s = lax.axis_index("s")
    bufs = [b0, b1]
    gsems = [g0, g1]
    osems = [o0, o1]
    base = s * (KJG * 128)
    pltpu.sync_copy(table.at[c, pl.ds(s * ROWS_PT, ROWS_PT)],
                    stg.at[pl.ds(s * ROWS_PT, ROWS_PT)])
    pltpu.sync_copy(idx.at[s], idx_v)
    plsc.subcore_barrier()

    def rnd(r, carry):
        j0 = r * NBUF
        gs = [pltpu.async_copy(stg.at[idx_v.at[j0 + b]], bufs[b], gsems[b])
              for b in range(NBUF)]
        os = []
        for b in range(NBUF):
            gs[b].wait()
            os.append(pltpu.async_copy(
                bufs[b], out.at[c, pl.ds(base + (j0 + b) * 128, 128)],
                osems[b]))
        for b in range(NBUF):
            os[b].wait()
        return carry

    lax.fori_loop(0, KJG // NBUF, rnd, 0)


@functools.lru_cache(maxsize=None)
def _gather_kernel():
    return pl.kernel(
        _gather_body,
        out_type=jax.ShapeDtypeStruct((NC, E_PAD, D), jnp.float32),
        mesh=_mesh(),
        scratch_types=[
            pltpu.VMEM((KJG, 128), jnp.int32),
            pltpu.VMEM((128, D), jnp.float32),
            pltpu.VMEM((128, D), jnp.float32),
            pltpu.VMEM_SHARED((N_PAD, D), jnp.float32),
        ] + [pltpu.SemaphoreType.DMA] * 4,
    )


def _sc_gather(table, idx3):
    return _gather_kernel()(table, idx3)


def _scatter_body(vals, idx, zeros, out, idx_v, b0, b1, acc,
                  l0, l1, a0, a1):
    c = lax.axis_index("c")
    s = lax.axis_index("s")
    wid = s * NC + c
    bufs = [b0, b1]
    lsems = [l0, l1]
    asems = [a0, a1]
    pltpu.sync_copy(idx.at[wid], idx_v)
    pltpu.sync_copy(zeros.at[pl.ds(s * ROWS_PT, ROWS_PT)],
                    acc.at[pl.ds(s * ROWS_PT, ROWS_PT)])
    plsc.subcore_barrier()

    def rnd(r, carry):
        j0 = r * NBUF
        ls = [pltpu.async_copy(
            vals.at[pl.ds(wid * BPW + (j0 + b) * 128, 128)], bufs[b],
            lsems[b]) for b in range(NBUF)]
        sc = []
        for b in range(NBUF):
            ls[b].wait()
            sc.append(pltpu.async_copy(bufs[b], acc.at[idx_v.at[j0 + b]],
                                       asems[b], add=True))
        for b in range(NBUF):
            sc[b].wait()
        return carry

    lax.fori_loop(0, KJ // NBUF, rnd, 0)
    plsc.subcore_barrier()
    pltpu.sync_copy(acc.at[pl.ds(s * ROWS_PT, ROWS_PT)],
                    out.at[c, pl.ds(s * ROWS_PT, ROWS_PT)])


@functools.lru_cache(maxsize=None)
def _scatter_kernel():
    return pl.kernel(
        _scatter_body,
        out_type=jax.ShapeDtypeStruct((NC, N_PAD, D), jnp.float32),
        mesh=_mesh(),
        scratch_types=[
            pltpu.VMEM((KJ, 128), jnp.int32),
            pltpu.VMEM((128, D), jnp.float32),
            pltpu.VMEM((128, D), jnp.float32),
            pltpu.VMEM_SHARED((N_PAD, D), jnp.float32),
        ] + [pltpu.SemaphoreType.DMA] * 4,
    )


def _sc_scatter(vals, idx3, zeros):
    return _scatter_kernel()(vals, idx3, zeros)


def _dot(a, b):
    return jnp.dot(a, b, preferred_element_type=jnp.float32)


def _b16(a):
    return a.astype(jnp.bfloat16).astype(jnp.float32)


_W = pl.BlockSpec((D, D), lambda i: (0, 0))
_B = pl.BlockSpec((1, D), lambda i: (0, 0))
_S = pl.BlockSpec((1, 1), lambda i: (0, 0))


def _edge_mlp_body(g_ref, s_ref, w2_ref, b2_ref, w3_ref, b3_ref, o_ref):
    i = pl.program_id(0)
    g = jnp.maximum(g_ref[0] + s_ref[...], 0.0)
    t = jnp.maximum(_dot(g, w2_ref[...]) + b2_ref[...], 0.0)
    o = _dot(t, w3_ref[...]) + b3_ref[...]
    rows = i * BE + lax.broadcasted_iota(jnp.int32, (BE, 1), 0)
    o_ref[...] = jnp.where(rows < E, o, 0.0)


def _edge_mlp(g, ss, w2, b2, w3, b3):
    eb = pl.BlockSpec((BE, D), lambda i: (i, 0))
    gb = pl.BlockSpec((1, BE, D), lambda i: (0, i, 0))
    return pl.pallas_call(
        _edge_mlp_body,
        grid=(E_PAD // BE,),
        in_specs=[gb, eb, _W, _B, _W, _B],
        out_specs=eb,
        out_shape=jax.ShapeDtypeStruct((E_PAD, D), jnp.float32),
    )(g, ss, w2, b2, w3, b3)


def _edge_clf_body(g_ref, s_ref, v1_ref, c1_ref, v2t_ref, c2_ref, ep_ref,
                   o_ref):
    g = jnp.maximum(g_ref[0] + s_ref[...], 0.0)
    t = jnp.maximum(_dot(g, v1_ref[...]) + c1_ref[...], 0.0)
    val = (jnp.sum(_b16(t) * _b16(v2t_ref[...]), axis=1, keepdims=True)
           + c2_ref[...])
    o_ref[...] = ep_ref[...] + val


def _edge_clf(g, ss, v1, c1, v2t, c2, ep):
    eb1 = pl.BlockSpec((BE, D), lambda i: (i, 1))
    gb1 = pl.BlockSpec((1, BE, D), lambda i: (1, i, 0))
    ev = pl.BlockSpec((BE, 1), lambda i: (i, 0))
    return pl.pallas_call(
        _edge_clf_body,
        grid=(E_PAD // BE,),
        in_specs=[gb1, eb1, _W, _B, _B, _S, ev],
        out_specs=ev,
        out_shape=jax.ShapeDtypeStruct((E_PAD, 1), jnp.float32),
    )(g, ss, v1, c1, v2t, c2, ep)


def _node_body(m_ref, h_ref, xn_ref, bnb_ref, a1_ref, a2_ref, np_ref,
               wm_ref, wh_ref, w1_ref, b1_ref, w2_ref, b2_ref,
               u0_ref, d0_ref, u1t_ref, d1_ref,
               wb1_ref, wd1_ref, vb1_ref, vd1_ref,
               h_o, cc_o, np_o):
    msg = m_ref[0] + m_ref[1]
    t = jnp.maximum(((_dot(msg, wm_ref[...]) + xn_ref[...])
                     + _dot(h_ref[...], wh_ref[...])) + bnb_ref[...], 0.0)
    t = jnp.maximum(_dot(t, w1_ref[...]) + b1_ref[...], 0.0)
    hn = _dot(t, w2_ref[...]) + b2_ref[...]
    u = jnp.maximum(_dot(hn, u0_ref[...]) + d0_ref[...], 0.0)
    npv = (jnp.sum(_b16(u) * _b16(u1t_ref[...]), axis=1, keepdims=True)
           + d1_ref[...])
    h_o[...] = hn
    cc_o[...] = jnp.stack(
        [a1_ref[...] + _dot(hn, wb1_ref[...]) + _dot(hn, wd1_ref[...]),
         a2_ref[...] + _dot(hn, vb1_ref[...]) + _dot(hn, vd1_ref[...])],
        axis=0)
    np_o[...] = np_ref[...] + npv


def _node_update(mp, h, xn, bn0, a1, a2, npd, wm, wh, w1, b1, w2, b2,
                 u0, d0, u1t, dd1, wb1, wd1, vb1, vd1):
    nb = pl.BlockSpec((BN, D), lambda i: (i, 0))
    nv = pl.BlockSpec((BN, 1), lambda i: (i, 0))
    return pl.pallas_call(
        _node_body,
        grid=(N // BN,),
        in_specs=[pl.BlockSpec((NC, BN, D), lambda i: (0, i, 0)),
                  nb, nb, _B, nb, nb, nv,
                  _W, _W, _W, _B, _W, _B, _W, _B, _B, _S, _W, _W, _W, _W],
        out_specs=[nb, pl.BlockSpec((NC, BN, D), lambda i: (0, i, 0)), nv],
        out_shape=[jax.ShapeDtypeStruct((N, D), jnp.float32),
                   jax.ShapeDtypeStruct((NC, N_PAD, D), jnp.float32),
                   jax.ShapeDtypeStruct((N, 1), jnp.float32)],
    )(mp, h, xn, bn0, a1, a2, npd, wm, wh, w1, b1, w2, b2,
      u0, d0, u1t, dd1, wb1, wd1, vb1, vd1)


def _pre_body(x_ref, p0, pb0, p1, pb1, p2, pb2,
              w0a, w0c, v0a, v0c, wnx, wb1, wd1,
              h_o, c1_o, a1_o, a2_o, bxs_o, xn_o):
    x = x_ref[...]
    t = jnp.maximum(_dot(x, p0[...]) + pb0[...], 0.0)
    t = jnp.maximum(_dot(t, p1[...]) + pb1[...], 0.0)
    h0 = _dot(t, p2[...]) + pb2[...]
    a1 = _dot(x, w0a[...])
    a2 = _dot(x, v0a[...])
    h_o[...] = h0
    c10 = a1 + _dot(h0, wb1[...]) + _dot(h0, wd1[...])
    c1_o[...] = jnp.stack([c10, c10], axis=0)
    a1_o[...] = a1
    a2_o[...] = a2
    bxs_o[...] = jnp.stack([_dot(x, w0c[...]), _dot(x, v0c[...])], axis=0)
    xn_o[...] = _dot(x, wnx[...])


def _precompute(x, p0, pb0, p1, pb1, p2, pb2, w0a, w0c, v0a, v0c, wnx,
                wb1, wd1):
    nb = pl.BlockSpec((BN, D), lambda i: (i, 0))
    return pl.pallas_call(
        _pre_body,
        grid=(N // BN,),
        in_specs=[nb, _W, _B, _W, _B, _W, _B, _W, _W, _W, _W, _W, _W, _W],
        out_specs=[nb, pl.BlockSpec((NC, BN, D), lambda i: (0, i, 0)), nb, nb,
                   pl.BlockSpec((NC, BN, D), lambda i: (0, i, 0)), nb],
        out_shape=[jax.ShapeDtypeStruct((N, D), jnp.float32),
                   jax.ShapeDtypeStruct((NC, N_PAD, D), jnp.float32),
                   jax.ShapeDtypeStruct((N, D), jnp.float32),
                   jax.ShapeDtypeStruct((N, D), jnp.float32),
                   jax.ShapeDtypeStruct((NC, N_PAD, D), jnp.float32),
                   jax.ShapeDtypeStruct((N, D), jnp.float32)],
    )(x, p0, pb0, p1, pb1, p2, pb2, w0a, w0c, v0a, v0c, wnx, wb1, wd1)


def _sassy_body(gs_ref, d_ref, wd1, b0, wd2, c0, ss_o):
    d = _b16(d_ref[...])
    s1 = (gs_ref[0] + d * _b16(wd1[...])) + b0[...]
    s2 = (gs_ref[1] + d * _b16(wd2[...])) + c0[...]
    ss_o[...] = jnp.concatenate([s1, s2], axis=1)


def _s_assembly(gsrc, d1, wd1, b0, wd2, c0):
    eb2 = pl.BlockSpec((BE, 2 * D), lambda i: (i, 0))
    gb2 = pl.BlockSpec((NC, BE, D), lambda i: (0, i, 0))
    ev = pl.BlockSpec((BE, 1), lambda i: (i, 0))
    return pl.pallas_call(
        _sassy_body,
        grid=(E_PAD // BE,),
        in_specs=[gb2, ev, _B, _B, _B, _B],
        out_specs=eb2,
        out_shape=jax.ShapeDtypeStruct((E_PAD, 2 * D), jnp.float32),
    )(gsrc, d1, wd1, b0, wd2, c0)


def kernel(node_features, distance, params, edge_index):
    x = node_features.astype(jnp.float32)
    src = edge_index[0].astype(jnp.int32)
    dst = edge_index[1].astype(jnp.int32)
    pad = E_PAD - E
    srcg = jnp.concatenate([src, jnp.zeros((pad,), jnp.int32)]) \
        .reshape(NS, KJG, 128)
    dstg = jnp.concatenate([dst, jnp.zeros((pad,), jnp.int32)]) \
        .reshape(NS, KJG, 128)
    dsts = dstg.reshape(NW, KJ, 128)
    d1 = jnp.concatenate([distance.astype(jnp.float32),
                          jnp.zeros((pad,), jnp.float32)]).reshape(E_PAD, 1)

    en = params['edge_net']
    nn = params['node_net']
    ec = params['edge_clf']
    ni = params['node_init']
    ncf = params['node_clf']

    w0 = en[0]['W']
    w0a, w0c = w0[0:D], w0[2 * D:3 * D]
    wb1, wdd1 = w0[D:2 * D], w0[3 * D:4 * D]
    wd1 = w0[4 * D:4 * D + 1]
    b0 = en[0]['b'].reshape(1, D)
    w2, b2 = en[1]['W'], en[1]['b'].reshape(1, D)
    w3, b3 = en[2]['W'], en[2]['b'].reshape(1, D)

    v0 = ec[0]['W']
    v0a, v0c = v0[0:D], v0[2 * D:3 * D]
    vb1, vdd1 = v0[D:2 * D], v0[3 * D:4 * D]
    wd2 = v0[4 * D:4 * D + 1]
    c0 = ec[0]['b'].reshape(1, D)
    v1, c1 = ec[1]['W'], ec[1]['b'].reshape(1, D)
    v2t = ec[2]['W'].reshape(1, D)
    c2 = ec[2]['b'].reshape(1, 1)

    wn0 = nn[0]['W']
    wm, wnx, wh = wn0[0:D], wn0[D:2 * D], wn0[2 * D:3 * D]
    bn0 = nn[0]['b'].reshape(1, D)
    wn1, bn1 = nn[1]['W'], nn[1]['b'].reshape(1, D)
    wn2, bn2 = nn[2]['W'], nn[2]['b'].reshape(1, D)

    u0, d0 = ncf[0]['W'], ncf[0]['b'].reshape(1, D)
    u1t = ncf[1]['W'].reshape(1, D)
    dd1 = ncf[1]['b'].reshape(1, 1)

    zeros_n = jnp.zeros((N_PAD, D), jnp.float32)
    ep = jnp.zeros((E_PAD, 1), jnp.float32)
    npd = jnp.zeros((N, 1), jnp.float32)

    h, c10, a1, a2, bxs, xn = _precompute(
        x, ni[0]['W'], ni[0]['b'].reshape(1, D), ni[1]['W'],
        ni[1]['b'].reshape(1, D), ni[2]['W'], ni[2]['b'].reshape(1, D),
        w0a, w0c, v0a, v0c, wnx, wb1, wdd1)
    gsrc = _sc_gather(bxs, srcg)
    ss = _s_assembly(gsrc, d1, wd1, b0, wd2, c0)
    gcur = _sc_gather(c10, dstg)

    for _ in range(N_ITERS):
        e3 = _edge_mlp(gcur, ss, w2, b2, w3, b3)
        mp = _sc_scatter(e3, dsts, zeros_n)
        h, cc, npd = _node_update(mp, h, xn, bn0, a1, a2, npd, wm, wh, wn1,
                                  bn1, wn2, bn2, u0, d0, u1t, dd1,
                                  wb1, wdd1, vb1, vdd1)
        gcur = _sc_gather(cc, dstg)
        ep = _edge_clf(gcur, ss, v1, c1, v2t, c2, ep)

    return h, ep[:E, 0], npd[:, 0]
